# Initial kernel scaffold; baseline (speedup 1.0000x reference)
import jax
import jax.numpy as jnp
from jax.experimental import pallas as pl


def kernel(x):
    raise NotImplementedError("write your pallas kernel here")


# speedup vs baseline: 1.0642x; 1.0642x over previous
import functools

import jax
import jax.numpy as jnp
from jax import lax
from jax.experimental import pallas as pl

_MASK_RATIO = 0.5
_L = 512
_ROWS_PER_STEP = 8
_BITS_HI = 0x3F800000


def _mask_fill_kernel(bits_ref, x_ref, out_ref, mask_ref, *, len_keep):
    bits = bits_ref[0]
    r = bits.shape[1]

    lo = jnp.zeros((1, r), jnp.int32)
    hi = jnp.full((1, r), _BITS_HI, jnp.int32)

    def body(_, carry):
        lo, hi = carry
        mid = (lo + hi) // 2
        cnt = jnp.sum((bits <= mid).astype(jnp.int32), axis=0, keepdims=True)
        pred = cnt >= len_keep
        hi = jnp.where(pred, mid, hi)
        lo = jnp.where(pred, lo, mid + 1)
        return lo, hi

    lo, hi = lax.fori_loop(0, 31, body, (lo, hi))
    t = lo

    cnt_lt = jnp.sum((bits < t).astype(jnp.float32), axis=0, keepdims=True)
    eq = bits == t

    row_ids = lax.broadcasted_iota(jnp.int32, (_L, _L), 0)
    col_ids = lax.broadcasted_iota(jnp.int32, (_L, _L), 1)
    tri = (col_ids < row_ids).astype(jnp.float32)
    eq_rank = jnp.dot(tri, eq.astype(jnp.float32),
                      preferred_element_type=jnp.float32)

    keep = (bits < t) | (eq & (cnt_lt + eq_rank < float(len_keep)))
    keepf = keep.astype(jnp.float32)
    mask_ref[0] = 1.0 - keepf
    for i in range(r):
        out_ref[i] = x_ref[i] * keepf[:, i:i + 1]


@jax.jit
def kernel(x):
    bs, nvars, L, D = x.shape
    len_keep = int(L * (1 - _MASK_RATIO))
    rows = bs * nvars

    noise = jax.random.uniform(jax.random.key(42), (bs, nvars, L),
                               dtype=jnp.float32)
    r = _ROWS_PER_STEP
    nblocks = rows // r
    bits3 = (lax.bitcast_convert_type(noise, jnp.int32)
             .reshape(nblocks, r, L).transpose(0, 2, 1))

    x3 = x.reshape(rows, L, D)
    grid = (nblocks,)

    out, maskf3 = pl.pallas_call(
        functools.partial(_mask_fill_kernel, len_keep=len_keep),
        grid=grid,
        in_specs=[
            pl.BlockSpec((1, L, r), lambda i: (i, 0, 0)),
            pl.BlockSpec((r, L, D), lambda i: (i, 0, 0)),
        ],
        out_specs=[
            pl.BlockSpec((r, L, D), lambda i: (i, 0, 0)),
            pl.BlockSpec((1, L, r), lambda i: (i, 0, 0)),
        ],
        out_shape=[
            jax.ShapeDtypeStruct((rows, L, D), x.dtype),
            jax.ShapeDtypeStruct((nblocks, L, r), jnp.float32),
        ],
    )(bits3, x3)

    x_mask = out.reshape(bs, nvars, L, D)
    mask = (maskf3.transpose(0, 2, 1).reshape(bs, nvars, L).astype(bool))
    return (x_mask, mask)


# speedup vs baseline: 1.3694x; 1.2868x over previous
import functools

import jax
import jax.numpy as jnp
from jax import lax
from jax.experimental import pallas as pl

_MASK_RATIO = 0.5
_L = 512
_ROWS_PER_STEP = 8
_BITS_HI = 0x3F800000


def _mask_kernel(bits_ref, keep_ref, mask_ref, *, len_keep):
    bits = bits_ref[...]
    rows = bits.shape[0]

    lo = jnp.zeros((rows, 1), jnp.int32)
    hi = jnp.full((rows, 1), _BITS_HI, jnp.int32)

    def body(_, carry):
        lo, hi = carry
        mid = (lo + hi) // 2
        cnt = jnp.sum((bits <= mid).astype(jnp.int32), axis=1, keepdims=True)
        pred = cnt >= len_keep
        hi = jnp.where(pred, mid, hi)
        lo = jnp.where(pred, lo, mid + 1)
        return lo, hi

    lo, hi = lax.fori_loop(0, 31, body, (lo, hi))
    t = lo

    cnt_lt = jnp.sum((bits < t).astype(jnp.float32), axis=1, keepdims=True)
    eq = bits == t

    row_ids = lax.broadcasted_iota(jnp.int32, (_L, _L), 0)
    col_ids = lax.broadcasted_iota(jnp.int32, (_L, _L), 1)
    tri = (row_ids < col_ids).astype(jnp.float32)
    eq_rank = jnp.dot(eq.astype(jnp.float32), tri,
                      preferred_element_type=jnp.float32)

    keep = (bits < t) | (eq & (cnt_lt + eq_rank < float(len_keep)))
    keepf = keep.astype(jnp.float32)
    keep_ref[...] = keepf
    mask_ref[...] = 1.0 - keepf


def _fill_kernel(keep_ref, x_ref, out_ref):
    r = x_ref.shape[0]
    keepf = keep_ref[0]
    for i in range(r):
        out_ref[i] = x_ref[i] * keepf[:, i:i + 1]


@jax.jit
def kernel(x):
    bs, nvars, L, D = x.shape
    len_keep = int(L * (1 - _MASK_RATIO))
    rows = bs * nvars

    noise = jax.random.uniform(jax.random.key(42), (bs, nvars, L),
                               dtype=jnp.float32)
    bits = lax.bitcast_convert_type(noise, jnp.int32).reshape(rows, L)

    keepf, maskf = pl.pallas_call(
        functools.partial(_mask_kernel, len_keep=len_keep),
        out_shape=[
            jax.ShapeDtypeStruct((rows, L), jnp.float32),
            jax.ShapeDtypeStruct((rows, L), jnp.float32),
        ],
    )(bits)

    r = _ROWS_PER_STEP
    nblocks = rows // r
    keep3 = keepf.reshape(nblocks, r, L).transpose(0, 2, 1)

    x3 = x.reshape(rows, L, D)
    out = pl.pallas_call(
        _fill_kernel,
        grid=(nblocks,),
        in_specs=[
            pl.BlockSpec((1, L, r), lambda i: (i, 0, 0)),
            pl.BlockSpec((r, L, D), lambda i: (i, 0, 0)),
        ],
        out_specs=pl.BlockSpec((r, L, D), lambda i: (i, 0, 0)),
        out_shape=jax.ShapeDtypeStruct((rows, L, D), x.dtype),
    )(keep3, x3)

    x_mask = out.reshape(bs, nvars, L, D)
    mask = maskf.reshape(bs, nvars, L).astype(bool)
    return (x_mask, mask)


# speedup vs baseline: 3.4446x; 2.5154x over previous
import functools

import jax
import jax.numpy as jnp
from jax import lax
from jax.experimental import pallas as pl

_MASK_RATIO = 0.5
_L = 512
_ROWS_PER_STEP = 8
_BITS_HI = 0x3F800000


def _mask_kernel(bits_ref, keep_ref, mask_ref, *, len_keep):
    bits = bits_ref[...]
    rows = bits.shape[0]

    lo = jnp.zeros((rows, 1), jnp.int32)
    hi = jnp.full((rows, 1), _BITS_HI, jnp.int32)

    def body(_, carry):
        lo, hi = carry
        mid = (lo + hi) // 2
        cnt = jnp.sum((bits <= mid).astype(jnp.int32), axis=1, keepdims=True)
        pred = cnt >= len_keep
        hi = jnp.where(pred, mid, hi)
        lo = jnp.where(pred, lo, mid + 1)
        return lo, hi

    lo, hi = lax.fori_loop(0, 31, body, (lo, hi))
    t = lo

    cnt_lt = jnp.sum((bits < t).astype(jnp.float32), axis=1, keepdims=True)
    eq = bits == t

    row_ids = lax.broadcasted_iota(jnp.int32, (_L, _L), 0)
    col_ids = lax.broadcasted_iota(jnp.int32, (_L, _L), 1)
    tri = (row_ids < col_ids).astype(jnp.float32)
    eq_rank = jnp.dot(eq.astype(jnp.float32), tri,
                      preferred_element_type=jnp.float32)

    keep = (bits < t) | (eq & (cnt_lt + eq_rank < float(len_keep)))
    keepf = keep.astype(jnp.float32)
    keep_ref[...] = keepf
    mask_ref[...] = 1.0 - keepf


def _fill_kernel(keep_ref, x_ref, out_ref):
    r = x_ref.shape[0]
    for i in range(r):
        out_ref[i] = x_ref[i] * keep_ref[0][i:i + 1]


@jax.jit
def kernel(x):
    bs, nvars, L, D = x.shape
    len_keep = int(L * (1 - _MASK_RATIO))
    rows = bs * nvars

    noise = jax.random.uniform(jax.random.key(42), (bs, nvars, L),
                               dtype=jnp.float32)
    bits = lax.bitcast_convert_type(noise, jnp.int32).reshape(rows, L)

    keepf, maskf = pl.pallas_call(
        functools.partial(_mask_kernel, len_keep=len_keep),
        out_shape=[
            jax.ShapeDtypeStruct((rows, L), jnp.float32),
            jax.ShapeDtypeStruct((rows, L), jnp.float32),
        ],
    )(bits)

    r = _ROWS_PER_STEP
    nblocks = rows // r
    keep3 = keepf.reshape(nblocks, r, L)

    xt = x.transpose(0, 1, 3, 2).reshape(rows, D, L)
    out = pl.pallas_call(
        _fill_kernel,
        grid=(nblocks,),
        in_specs=[
            pl.BlockSpec((1, r, L), lambda i: (i, 0, 0)),
            pl.BlockSpec((r, D, L), lambda i: (i, 0, 0)),
        ],
        out_specs=pl.BlockSpec((r, D, L), lambda i: (i, 0, 0)),
        out_shape=jax.ShapeDtypeStruct((rows, D, L), x.dtype),
    )(keep3, xt)

    x_mask = out.reshape(bs, nvars, D, L).transpose(0, 1, 3, 2)
    mask = maskf.reshape(bs, nvars, L).astype(bool)
    return (x_mask, mask)


# speedup vs baseline: 4.2701x; 1.2396x over previous
import functools

import jax
import jax.numpy as jnp
from jax import lax
from jax.experimental import pallas as pl

_MASK_RATIO = 0.5
_L = 512
_ROWS_PER_STEP = 16
_BITS_HI = 0x3F800000


def _mask_kernel(bits_ref, keep_ref, mask_ref, *, len_keep):
    bits = bits_ref[...]
    rows = bits.shape[0]

    lo = jnp.zeros((rows, 1), jnp.int32)
    hi = jnp.full((rows, 1), _BITS_HI, jnp.int32)

    def body(_, carry):
        lo, hi = carry
        mid = (lo + hi) // 2
        cnt = jnp.sum((bits <= mid).astype(jnp.int32), axis=1, keepdims=True)
        pred = cnt >= len_keep
        hi = jnp.where(pred, mid, hi)
        lo = jnp.where(pred, lo, mid + 1)
        return lo, hi

    lo, hi = lax.fori_loop(0, 31, body, (lo, hi))
    t = lo

    cnt_lt = jnp.sum((bits < t).astype(jnp.float32), axis=1, keepdims=True)
    eq = bits == t

    row_ids = lax.broadcasted_iota(jnp.int32, (_L, _L), 0)
    col_ids = lax.broadcasted_iota(jnp.int32, (_L, _L), 1)
    tri = (row_ids < col_ids).astype(jnp.float32)
    eq_rank = jnp.dot(eq.astype(jnp.float32), tri,
                      preferred_element_type=jnp.float32)

    keep = (bits < t) | (eq & (cnt_lt + eq_rank < float(len_keep)))
    keepf = keep.astype(jnp.float32)
    keep_ref[...] = keepf
    mask_ref[...] = 1.0 - keepf


def _fill_kernel(keep_ref, x_ref, out_ref):
    r = x_ref.shape[0]
    for i in range(r):
        out_ref[i] = x_ref[i] * keep_ref[0][i:i + 1]


@jax.jit
def kernel(x):
    bs, nvars, L, D = x.shape
    len_keep = int(L * (1 - _MASK_RATIO))
    rows = bs * nvars

    noise = jax.random.uniform(jax.random.key(42), (bs, nvars, L),
                               dtype=jnp.float32)
    bits = lax.bitcast_convert_type(noise, jnp.int32).reshape(rows, L)

    keepf, maskf = pl.pallas_call(
        functools.partial(_mask_kernel, len_keep=len_keep),
        out_shape=[
            jax.ShapeDtypeStruct((rows, L), jnp.float32),
            jax.ShapeDtypeStruct((rows, L), jnp.float32),
        ],
    )(bits)

    r = _ROWS_PER_STEP
    nblocks = rows // r
    keep3 = keepf.reshape(nblocks, r, L)

    xt = x.transpose(0, 1, 3, 2).reshape(rows, D, L)
    out = pl.pallas_call(
        _fill_kernel,
        grid=(nblocks,),
        in_specs=[
            pl.BlockSpec((1, r, L), lambda i: (i, 0, 0)),
            pl.BlockSpec((r, D, L), lambda i: (i, 0, 0)),
        ],
        out_specs=pl.BlockSpec((r, D, L), lambda i: (i, 0, 0)),
        out_shape=jax.ShapeDtypeStruct((rows, D, L), x.dtype),
    )(keep3, xt)

    x_mask = out.reshape(bs, nvars, D, L).transpose(0, 1, 3, 2)
    mask = maskf.reshape(bs, nvars, L).astype(bool)
    return (x_mask, mask)


# speedup vs baseline: 4.5074x; 1.0556x over previous
import functools

import jax
import jax.numpy as jnp
from jax import lax
from jax.experimental import pallas as pl

_MASK_RATIO = 0.5
_L = 512
_ROWS_PER_STEP = 32
_BITS_HI = 0x3F800000


def _mask_kernel(bits_ref, keep_ref, mask_ref, *, len_keep):
    bits = bits_ref[...]
    rows = bits.shape[0]

    lo = jnp.zeros((rows, 1), jnp.int32)
    hi = jnp.full((rows, 1), _BITS_HI, jnp.int32)

    def body(_, carry):
        lo, hi = carry
        mid = (lo + hi) // 2
        cnt = jnp.sum((bits <= mid).astype(jnp.int32), axis=1, keepdims=True)
        pred = cnt >= len_keep
        hi = jnp.where(pred, mid, hi)
        lo = jnp.where(pred, lo, mid + 1)
        return lo, hi

    lo, hi = lax.fori_loop(0, 31, body, (lo, hi))
    t = lo

    cnt_lt = jnp.sum((bits < t).astype(jnp.float32), axis=1, keepdims=True)
    eq = bits == t

    row_ids = lax.broadcasted_iota(jnp.int32, (_L, _L), 0)
    col_ids = lax.broadcasted_iota(jnp.int32, (_L, _L), 1)
    tri = (row_ids < col_ids).astype(jnp.float32)
    eq_rank = jnp.dot(eq.astype(jnp.float32), tri,
                      preferred_element_type=jnp.float32)

    keep = (bits < t) | (eq & (cnt_lt + eq_rank < float(len_keep)))
    keepf = keep.astype(jnp.float32)
    keep_ref[...] = keepf
    mask_ref[...] = 1.0 - keepf


def _fill_kernel(keep_ref, x_ref, out_ref):
    r = x_ref.shape[0]
    for i in range(r):
        out_ref[i] = x_ref[i] * keep_ref[0][i:i + 1]


@jax.jit
def kernel(x):
    bs, nvars, L, D = x.shape
    len_keep = int(L * (1 - _MASK_RATIO))
    rows = bs * nvars

    noise = jax.random.uniform(jax.random.key(42), (bs, nvars, L),
                               dtype=jnp.float32)
    bits = lax.bitcast_convert_type(noise, jnp.int32).reshape(rows, L)

    keepf, maskf = pl.pallas_call(
        functools.partial(_mask_kernel, len_keep=len_keep),
        out_shape=[
            jax.ShapeDtypeStruct((rows, L), jnp.float32),
            jax.ShapeDtypeStruct((rows, L), jnp.float32),
        ],
    )(bits)

    r = _ROWS_PER_STEP
    nblocks = rows // r
    keep3 = keepf.reshape(nblocks, r, L)

    xt = x.transpose(0, 1, 3, 2).reshape(rows, D, L)
    out = pl.pallas_call(
        _fill_kernel,
        grid=(nblocks,),
        in_specs=[
            pl.BlockSpec((1, r, L), lambda i: (i, 0, 0)),
            pl.BlockSpec((r, D, L), lambda i: (i, 0, 0)),
        ],
        out_specs=pl.BlockSpec((r, D, L), lambda i: (i, 0, 0)),
        out_shape=jax.ShapeDtypeStruct((rows, D, L), x.dtype),
    )(keep3, xt)

    x_mask = out.reshape(bs, nvars, D, L).transpose(0, 1, 3, 2)
    mask = maskf.reshape(bs, nvars, L).astype(bool)
    return (x_mask, mask)


# speedup vs baseline: 4.5872x; 1.0177x over previous
import functools

import jax
import jax.numpy as jnp
from jax import lax
from jax.experimental import pallas as pl

_MASK_RATIO = 0.5
_L = 512
_ROWS_PER_STEP = 64
_BITS_HI = 0x3F800000


def _mask_kernel(bits_ref, keep_ref, mask_ref, *, len_keep):
    bits = bits_ref[...]
    rows = bits.shape[0]

    lo = jnp.zeros((rows, 1), jnp.int32)
    hi = jnp.full((rows, 1), _BITS_HI, jnp.int32)

    def body(_, carry):
        lo, hi = carry
        mid = (lo + hi) // 2
        cnt = jnp.sum((bits <= mid).astype(jnp.int32), axis=1, keepdims=True)
        pred = cnt >= len_keep
        hi = jnp.where(pred, mid, hi)
        lo = jnp.where(pred, lo, mid + 1)
        return lo, hi

    lo, hi = lax.fori_loop(0, 31, body, (lo, hi))
    t = lo

    cnt_lt = jnp.sum((bits < t).astype(jnp.float32), axis=1, keepdims=True)
    eq = bits == t

    row_ids = lax.broadcasted_iota(jnp.int32, (_L, _L), 0)
    col_ids = lax.broadcasted_iota(jnp.int32, (_L, _L), 1)
    tri = (row_ids < col_ids).astype(jnp.float32)
    eq_rank = jnp.dot(eq.astype(jnp.float32), tri,
                      preferred_element_type=jnp.float32)

    keep = (bits < t) | (eq & (cnt_lt + eq_rank < float(len_keep)))
    keepf = keep.astype(jnp.float32)
    keep_ref[...] = keepf
    mask_ref[...] = 1.0 - keepf


def _fill_kernel(keep_ref, x_ref, out_ref):
    r = x_ref.shape[0]
    for i in range(r):
        out_ref[i] = x_ref[i] * keep_ref[0][i:i + 1]


@jax.jit
def kernel(x):
    bs, nvars, L, D = x.shape
    len_keep = int(L * (1 - _MASK_RATIO))
    rows = bs * nvars

    noise = jax.random.uniform(jax.random.key(42), (bs, nvars, L),
                               dtype=jnp.float32)
    bits = lax.bitcast_convert_type(noise, jnp.int32).reshape(rows, L)

    keepf, maskf = pl.pallas_call(
        functools.partial(_mask_kernel, len_keep=len_keep),
        out_shape=[
            jax.ShapeDtypeStruct((rows, L), jnp.float32),
            jax.ShapeDtypeStruct((rows, L), jnp.float32),
        ],
    )(bits)

    r = _ROWS_PER_STEP
    nblocks = rows // r
    keep3 = keepf.reshape(nblocks, r, L)

    xt = x.transpose(0, 1, 3, 2).reshape(rows, D, L)
    out = pl.pallas_call(
        _fill_kernel,
        grid=(nblocks,),
        in_specs=[
            pl.BlockSpec((1, r, L), lambda i: (i, 0, 0)),
            pl.BlockSpec((r, D, L), lambda i: (i, 0, 0)),
        ],
        out_specs=pl.BlockSpec((r, D, L), lambda i: (i, 0, 0)),
        out_shape=jax.ShapeDtypeStruct((rows, D, L), x.dtype),
    )(keep3, xt)

    x_mask = out.reshape(bs, nvars, D, L).transpose(0, 1, 3, 2)
    mask = maskf.reshape(bs, nvars, L).astype(bool)
    return (x_mask, mask)
